# baseline (speedup 1.0000x reference)
import jax
import jax.numpy as jnp
from jax.experimental import pallas as pl


def _copy_body(in_ref, out_ref):
    out_ref[...] = in_ref[...]


def kernel(x):
    M, N = x.shape
    BM = 512
    gathered = pl.pallas_call(
        _copy_body,
        grid=(M // BM,),
        in_specs=[pl.BlockSpec((BM, N), lambda i: (i, 0))],
        out_specs=pl.BlockSpec((BM, N), lambda i: (i, 0)),
        out_shape=jax.ShapeDtypeStruct((M, N), x.dtype),
    )(x)
    sizes = jnp.asarray([M], dtype=jnp.int32)
    return (gathered, sizes)
